# baseline (speedup 1.0000x reference)
import functools

import jax
import jax.numpy as jnp
from jax import lax
from jax.experimental import pallas as pl
from jax.experimental.pallas import tpu as pltpu
from jax.experimental.pallas import tpu_sc as plsc

N = 10000
D = 128
E = 320000

N_PAD = 10240
ROW_BLK = 512
NW = 32
CHUNK = 128
CH_PER_W = 80
E_PAD = NW * CH_PER_W * CHUNK
PER_TILE = N_PAD // 16


NBUF = 2
CH_HALF = 16
H0 = 10
H1 = 0
NCH = 16 * (H0 + H1) * CH_HALF


def _sc_agg_body(x_hbm, srcm_hbm, dstm_hbm, out_hbm,
                 src_v, dst_v, rows, acc, g0, g1, s0, s1):
    gs = (g0, g1)
    ss = (s0, s1)
    cid = lax.axis_index("c")
    sid = lax.axis_index("s")
    my_halves = jnp.where(cid == 0, H0, H1)
    chunk0 = jnp.where(cid == 0, sid * (H0 * CH_HALF),
                       16 * H0 * CH_HALF + sid * (H1 * CH_HALF))

    zero16 = jnp.zeros((16,), jnp.float32)

    def _zrow(i, carry):
        for t in range(8):
            rows[0, i, pl.ds(t * 16, 16)] = zero16
        return carry

    base = sid * PER_TILE
    nz = PER_TILE // CHUNK

    with jax.named_scope("ph_zero"):
        lax.fori_loop(0, CHUNK, _zrow, 0)
        for k in range(nz):
            pltpu.async_copy(rows.at[0],
                             acc.at[pl.ds(base + k * CHUNK, CHUNK)],
                             gs[k % NBUF])
        for k in range(nz):
            pltpu.make_async_copy(rows.at[0],
                                  acc.at[pl.ds(base + k * CHUNK, CHUNK)],
                                  gs[k % NBUF]).wait()

        @pl.when(cid == 1)
        def _early_out():
            pltpu.sync_copy(acc.at[pl.ds(base, PER_TILE)],
                            out_hbm.at[pl.ds(N_PAD + base, PER_TILE)])

        plsc.subcore_barrier()

    for h in range(max(H0, H1)):
      with jax.named_scope(f"ph_loop{h}"):
       @pl.when(h < my_halves)
       def _run_half(h=h):
        pltpu.sync_copy(
            srcm_hbm.at[pl.ds(chunk0 + h * CH_HALF, CH_HALF)], src_v)
        pltpu.sync_copy(
            dstm_hbm.at[pl.ds(chunk0 + h * CH_HALF, CH_HALF)], dst_v)

        def _group(jj, carry):
            jbase = jj * NBUF
            for t in range(NBUF):
                @pl.when(jj > 0)
                def _wait_prev_scatter(t=t, jbase=jbase):
                    pltpu.make_async_copy(
                        rows.at[t], acc.at[dst_v.at[jbase + t]], ss[t]).wait()
                pltpu.async_copy(x_hbm.at[src_v.at[jbase + t]], rows.at[t],
                                 gs[t])
            for t in range(NBUF):
                pltpu.make_async_copy(x_hbm.at[src_v.at[jbase + t]],
                                      rows.at[t], gs[t]).wait()
                pltpu.async_copy(rows.at[t], acc.at[dst_v.at[jbase + t]],
                                 ss[t], add=True)
            return carry

        lax.fori_loop(0, CH_HALF // NBUF, _group, 0)
        for t in range(NBUF):
            pltpu.make_async_copy(rows.at[t],
                                  acc.at[dst_v.at[CH_HALF - NBUF + t]],
                                  ss[t]).wait()

    with jax.named_scope("ph_out"):
        plsc.subcore_barrier()

        @pl.when(cid == 0)
        def _copy_out():
            pltpu.sync_copy(acc.at[pl.ds(base, PER_TILE)],
                            out_hbm.at[pl.ds(base, PER_TILE)])


def _sc_agg(x, srcm, dstm):
    mesh = plsc.VectorSubcoreMesh(core_axis_name="c", subcore_axis_name="s",
                                  num_cores=2, num_subcores=16)
    return pl.kernel(
        _sc_agg_body,
        out_type=jax.ShapeDtypeStruct((2 * N_PAD, D), jnp.float32),
        mesh=mesh,
        scratch_types=[
            pltpu.VMEM((CH_HALF, CHUNK), jnp.int32),
            pltpu.VMEM((CH_HALF, CHUNK), jnp.int32),
            pltpu.VMEM((NBUF, CHUNK, D), jnp.float32),
            pltpu.VMEM_SHARED((N_PAD, D), jnp.float32),
            pltpu.SemaphoreType.DMA,
            pltpu.SemaphoreType.DMA,
            pltpu.SemaphoreType.DMA,
            pltpu.SemaphoreType.DMA,
        ],
    )(x, srcm, dstm)


def _mlp_conv_body(x_ref, p0_ref, p1_ref, wa_ref, ba_ref, wb_ref, bb_ref,
                   o_ref):
    z = x_ref[...] + p0_ref[...] + p1_ref[...]
    z = jnp.maximum(
        jnp.dot(z, wa_ref[...], preferred_element_type=jnp.float32)
        + ba_ref[...], 0.0)
    z = jnp.dot(z, wb_ref[...], preferred_element_type=jnp.float32) + bb_ref[...]
    o_ref[...] = jnp.maximum(z, 0.0)


def _mlp_head_body(x_ref, p0_ref, p1_ref, wa_ref, ba_ref, wb_ref, bb_ref,
                   wf1_ref, bf1_ref, wf2_ref, bf2_ref, o_ref):
    z = x_ref[...] + p0_ref[...] + p1_ref[...]
    z = jnp.maximum(
        jnp.dot(z, wa_ref[...], preferred_element_type=jnp.float32)
        + ba_ref[...], 0.0)
    z = jnp.dot(z, wb_ref[...], preferred_element_type=jnp.float32) + bb_ref[...]
    z = jnp.maximum(z, 0.0)
    z = jnp.maximum(
        jnp.dot(z, wf1_ref[...], preferred_element_type=jnp.float32)
        + bf1_ref[...], 0.0)
    o_ref[...] = (jnp.dot(z, wf2_ref[...], preferred_element_type=jnp.float32)
                  + bf2_ref[...])


def _row_spec():
    return pl.BlockSpec((ROW_BLK, D), lambda i: (i, 0))


def _p_specs():
    off = N_PAD // ROW_BLK
    return (pl.BlockSpec((ROW_BLK, D), lambda i: (i, 0)),
            pl.BlockSpec((ROW_BLK, D), lambda i, _o=off: (i + _o, 0)))


def _w_spec():
    return pl.BlockSpec((D, D), lambda i: (0, 0))


def _b_spec():
    return pl.BlockSpec((1, D), lambda i: (0, 0))


def _mlp_conv(x, parts, wa, ba, wb, bb):
    p0s, p1s = _p_specs()
    return pl.pallas_call(
        _mlp_conv_body,
        grid=(N_PAD // ROW_BLK,),
        in_specs=[_row_spec(), p0s, p1s, _w_spec(), _b_spec(), _w_spec(),
                  _b_spec()],
        out_specs=_row_spec(),
        out_shape=jax.ShapeDtypeStruct((N_PAD, D), jnp.float32),
    )(x, parts, parts, wa, ba.reshape(1, D), wb, bb.reshape(1, D))


def _mlp_head(x, parts, wa, ba, wb, bb, wf1, bf1, wf2p, bf2p):
    p0s, p1s = _p_specs()
    return pl.pallas_call(
        _mlp_head_body,
        grid=(N_PAD // ROW_BLK,),
        in_specs=[_row_spec(), p0s, p1s, _w_spec(), _b_spec(), _w_spec(),
                  _b_spec(), _w_spec(), _b_spec(), _w_spec(), _b_spec()],
        out_specs=_row_spec(),
        out_shape=jax.ShapeDtypeStruct((N_PAD, D), jnp.float32),
    )(x, parts, parts, wa, ba.reshape(1, D), wb, bb.reshape(1, D),
      wf1, bf1.reshape(1, D), wf2p, bf2p.reshape(1, D))


def kernel(h, bf, edge_index, edge_weight, W1a, b1a, W1b, b1b, W2a, b2a,
           W2b, b2b, Wf1, bf1, Wf2, bf2):
    src = edge_index[0].astype(jnp.int32)
    dst = edge_index[1].astype(jnp.int32)
    pad = E_PAD - E
    srcm = jnp.concatenate([src, jnp.zeros((pad,), jnp.int32)]
                           ).reshape(E_PAD // CHUNK, CHUNK)
    dummy = N + (jnp.arange(pad, dtype=jnp.int32) % (N_PAD - N))
    dstm = jnp.concatenate([dst, dummy]).reshape(E_PAD // CHUNK, CHUNK)

    wf2p = jnp.pad(Wf2, ((0, 0), (0, D - 2)))
    bf2p = jnp.pad(bf2, (0, D - 2))

    h_pad = jnp.pad(h, ((0, N_PAD - N), (0, 0)))
    parts1 = _sc_agg(h, srcm, dstm)
    x1 = _mlp_conv(h_pad, parts1, W1a, b1a, W1b, b1b)
    parts2 = _sc_agg(x1, srcm, dstm)
    out_pad = _mlp_head(x1, parts2, W2a, b2a, W2b, b2b, Wf1, bf1, wf2p, bf2p)
    return out_pad[:N, :2]


# speedup vs baseline: 1.5069x; 1.5069x over previous
import functools

import jax
import jax.numpy as jnp
from jax import lax
from jax.experimental import pallas as pl
from jax.experimental.pallas import tpu as pltpu
from jax.experimental.pallas import tpu_sc as plsc

N = 10000
D = 128
E = 320000

N_PAD = 10240
ROW_BLK = 512
NW = 32
CHUNK = 128
CH_PER_W = 80
E_PAD = NW * CH_PER_W * CHUNK
PER_TILE = N_PAD // 16


NBUF = 2
CH_HALF = 16
H0 = 9
H1 = 1
NCH = 16 * (H0 + H1) * CH_HALF


def _sc_agg_body(x_hbm, srcm_hbm, dstm_hbm, out_hbm,
                 src_v, dst_v, rows, acc, g0, g1, s0, s1):
    gs = (g0, g1)
    ss = (s0, s1)
    cid = lax.axis_index("c")
    sid = lax.axis_index("s")
    my_halves = jnp.where(cid == 0, H0, H1)
    chunk0 = jnp.where(cid == 0, sid * (H0 * CH_HALF),
                       16 * H0 * CH_HALF + sid * (H1 * CH_HALF))

    zero16 = jnp.zeros((16,), jnp.float32)

    def _zrow(i, carry):
        for t in range(8):
            rows[0, i, pl.ds(t * 16, 16)] = zero16
        return carry

    base = sid * PER_TILE
    nz = PER_TILE // CHUNK

    with jax.named_scope("ph_zero"):
        lax.fori_loop(0, CHUNK, _zrow, 0)
        for k in range(nz):
            pltpu.async_copy(rows.at[0],
                             acc.at[pl.ds(base + k * CHUNK, CHUNK)],
                             gs[k % NBUF])
        for k in range(nz):
            pltpu.make_async_copy(rows.at[0],
                                  acc.at[pl.ds(base + k * CHUNK, CHUNK)],
                                  gs[k % NBUF]).wait()

        plsc.subcore_barrier()

    for h in range(max(H0, H1)):
      with jax.named_scope(f"ph_loop{h}"):
       @pl.when(h < my_halves)
       def _run_half(h=h):
        pltpu.sync_copy(
            srcm_hbm.at[pl.ds(chunk0 + h * CH_HALF, CH_HALF)], src_v)
        pltpu.sync_copy(
            dstm_hbm.at[pl.ds(chunk0 + h * CH_HALF, CH_HALF)], dst_v)

        def _group(jj, carry):
            jbase = jj * NBUF
            for t in range(NBUF):
                @pl.when(jj > 0)
                def _wait_prev_scatter(t=t, jbase=jbase):
                    pltpu.make_async_copy(
                        rows.at[t], acc.at[dst_v.at[jbase + t]], ss[t]).wait()
                pltpu.async_copy(x_hbm.at[src_v.at[jbase + t]], rows.at[t],
                                 gs[t])
            for t in range(NBUF):
                pltpu.make_async_copy(x_hbm.at[src_v.at[jbase + t]],
                                      rows.at[t], gs[t]).wait()
                pltpu.async_copy(rows.at[t], acc.at[dst_v.at[jbase + t]],
                                 ss[t], add=True)
            return carry

        lax.fori_loop(0, CH_HALF // NBUF, _group, 0)
        for t in range(NBUF):
            pltpu.make_async_copy(rows.at[t],
                                  acc.at[dst_v.at[CH_HALF - NBUF + t]],
                                  ss[t]).wait()

    with jax.named_scope("ph_out"):
        plsc.subcore_barrier()

        pltpu.sync_copy(acc.at[pl.ds(base, PER_TILE)],
                        out_hbm.at[pl.ds(cid * N_PAD + base, PER_TILE)])


def _sc_agg(x, srcm, dstm):
    mesh = plsc.VectorSubcoreMesh(core_axis_name="c", subcore_axis_name="s",
                                  num_cores=2, num_subcores=16)
    return pl.kernel(
        _sc_agg_body,
        out_type=jax.ShapeDtypeStruct((2 * N_PAD, D), jnp.float32),
        mesh=mesh,
        scratch_types=[
            pltpu.VMEM((CH_HALF, CHUNK), jnp.int32),
            pltpu.VMEM((CH_HALF, CHUNK), jnp.int32),
            pltpu.VMEM((NBUF, CHUNK, D), jnp.float32),
            pltpu.VMEM_SHARED((N_PAD, D), jnp.float32),
            pltpu.SemaphoreType.DMA,
            pltpu.SemaphoreType.DMA,
            pltpu.SemaphoreType.DMA,
            pltpu.SemaphoreType.DMA,
        ],
    )(x, srcm, dstm)


def _mlp_conv_body(x_ref, p0_ref, p1_ref, wa_ref, ba_ref, wb_ref, bb_ref,
                   o_ref):
    z = x_ref[...] + p0_ref[...] + p1_ref[...]
    z = jnp.maximum(
        jnp.dot(z, wa_ref[...], preferred_element_type=jnp.float32)
        + ba_ref[...], 0.0)
    z = jnp.dot(z, wb_ref[...], preferred_element_type=jnp.float32) + bb_ref[...]
    o_ref[...] = jnp.maximum(z, 0.0)


def _mlp_head_body(x_ref, p0_ref, p1_ref, wa_ref, ba_ref, wb_ref, bb_ref,
                   wf1_ref, bf1_ref, wf2_ref, bf2_ref, o_ref):
    z = x_ref[...] + p0_ref[...] + p1_ref[...]
    z = jnp.maximum(
        jnp.dot(z, wa_ref[...], preferred_element_type=jnp.float32)
        + ba_ref[...], 0.0)
    z = jnp.dot(z, wb_ref[...], preferred_element_type=jnp.float32) + bb_ref[...]
    z = jnp.maximum(z, 0.0)
    z = jnp.maximum(
        jnp.dot(z, wf1_ref[...], preferred_element_type=jnp.float32)
        + bf1_ref[...], 0.0)
    o_ref[...] = (jnp.dot(z, wf2_ref[...], preferred_element_type=jnp.float32)
                  + bf2_ref[...])


def _row_spec():
    return pl.BlockSpec((ROW_BLK, D), lambda i: (i, 0))


def _p_specs():
    off = N_PAD // ROW_BLK
    return (pl.BlockSpec((ROW_BLK, D), lambda i: (i, 0)),
            pl.BlockSpec((ROW_BLK, D), lambda i, _o=off: (i + _o, 0)))


def _w_spec():
    return pl.BlockSpec((D, D), lambda i: (0, 0))


def _b_spec():
    return pl.BlockSpec((1, D), lambda i: (0, 0))


def _mlp_conv(x, parts, wa, ba, wb, bb):
    p0s, p1s = _p_specs()
    return pl.pallas_call(
        _mlp_conv_body,
        grid=(N_PAD // ROW_BLK,),
        in_specs=[_row_spec(), p0s, p1s, _w_spec(), _b_spec(), _w_spec(),
                  _b_spec()],
        out_specs=_row_spec(),
        out_shape=jax.ShapeDtypeStruct((N_PAD, D), jnp.float32),
    )(x, parts, parts, wa, ba.reshape(1, D), wb, bb.reshape(1, D))


def _mlp_head(x, parts, wa, ba, wb, bb, wf1, bf1, wf2p, bf2p):
    p0s, p1s = _p_specs()
    return pl.pallas_call(
        _mlp_head_body,
        grid=(N_PAD // ROW_BLK,),
        in_specs=[_row_spec(), p0s, p1s, _w_spec(), _b_spec(), _w_spec(),
                  _b_spec(), _w_spec(), _b_spec(), _w_spec(), _b_spec()],
        out_specs=_row_spec(),
        out_shape=jax.ShapeDtypeStruct((N_PAD, D), jnp.float32),
    )(x, parts, parts, wa, ba.reshape(1, D), wb, bb.reshape(1, D),
      wf1, bf1.reshape(1, D), wf2p, bf2p.reshape(1, D))


def kernel(h, bf, edge_index, edge_weight, W1a, b1a, W1b, b1b, W2a, b2a,
           W2b, b2b, Wf1, bf1, Wf2, bf2):
    src = edge_index[0].astype(jnp.int32)
    dst = edge_index[1].astype(jnp.int32)
    pad = E_PAD - E
    srcm = jnp.concatenate([src, jnp.zeros((pad,), jnp.int32)]
                           ).reshape(E_PAD // CHUNK, CHUNK)
    dummy = N + (jnp.arange(pad, dtype=jnp.int32) % (N_PAD - N))
    dstm = jnp.concatenate([dst, dummy]).reshape(E_PAD // CHUNK, CHUNK)

    wf2p = jnp.pad(Wf2, ((0, 0), (0, D - 2)))
    bf2p = jnp.pad(bf2, (0, D - 2))

    h_pad = jnp.pad(h, ((0, N_PAD - N), (0, 0)))
    parts1 = _sc_agg(h, srcm, dstm)
    x1 = _mlp_conv(h_pad, parts1, W1a, b1a, W1b, b1b)
    parts2 = _sc_agg(x1, srcm, dstm)
    out_pad = _mlp_head(x1, parts2, W2a, b2a, W2b, b2b, Wf1, bf1, wf2p, bf2p)
    return out_pad[:N, :2]


# speedup vs baseline: 1.5265x; 1.0130x over previous
import jax
import jax.numpy as jnp
from jax import lax
from jax.experimental import pallas as pl
from jax.experimental.pallas import tpu as pltpu
from jax.experimental.pallas import tpu_sc as plsc

N = 10000
D = 128
E = 320000

N_PAD = 10240
ROW_BLK = 512
NW = 32
CHUNK = 128
CH_PER_W = 80
E_PAD = NW * CH_PER_W * CHUNK
PER_TILE = N_PAD // 16


NBUF = 2
CH_HALF = 8
H0 = 19
H1 = 1
NCH = 16 * (H0 + H1) * CH_HALF


def _sc_agg_body(x_hbm, srcm_hbm, dstm_hbm, out_hbm,
                 src_v, dst_v, rows, acc, g0, g1, s0, s1):
    gs = (g0, g1)
    ss = (s0, s1)
    cid = lax.axis_index("c")
    sid = lax.axis_index("s")
    my_halves = jnp.where(cid == 0, H0, H1)
    chunk0 = jnp.where(cid == 0, sid * (H0 * CH_HALF),
                       16 * H0 * CH_HALF + sid * (H1 * CH_HALF))

    zero16 = jnp.zeros((16,), jnp.float32)

    def _zrow(i, carry):
        for t in range(8):
            rows[0, i, pl.ds(t * 16, 16)] = zero16
        return carry

    base = sid * PER_TILE
    nz = PER_TILE // CHUNK

    with jax.named_scope("ph_zero"):
        lax.fori_loop(0, CHUNK, _zrow, 0)
        for k in range(nz):
            pltpu.async_copy(rows.at[0],
                             acc.at[pl.ds(base + k * CHUNK, CHUNK)],
                             gs[k % NBUF])
        for k in range(nz):
            pltpu.make_async_copy(rows.at[0],
                                  acc.at[pl.ds(base + k * CHUNK, CHUNK)],
                                  gs[k % NBUF]).wait()

        plsc.subcore_barrier()

    for h in range(max(H0, H1)):
      with jax.named_scope(f"ph_loop{h}"):
       @pl.when(h < my_halves)
       def _run_half(h=h):
        pltpu.sync_copy(
            srcm_hbm.at[pl.ds(chunk0 + h * CH_HALF, CH_HALF)], src_v)
        pltpu.sync_copy(
            dstm_hbm.at[pl.ds(chunk0 + h * CH_HALF, CH_HALF)], dst_v)

        def _group(jj, carry):
            jbase = jj * NBUF
            for t in range(NBUF):
                @pl.when(jj > 0)
                def _wait_prev_scatter(t=t, jbase=jbase):
                    pltpu.make_async_copy(
                        rows.at[t], acc.at[dst_v.at[jbase + t]], ss[t]).wait()
                pltpu.async_copy(x_hbm.at[src_v.at[jbase + t]], rows.at[t],
                                 gs[t])
            for t in range(NBUF):
                pltpu.make_async_copy(x_hbm.at[src_v.at[jbase + t]],
                                      rows.at[t], gs[t]).wait()
                pltpu.async_copy(rows.at[t], acc.at[dst_v.at[jbase + t]],
                                 ss[t], add=True)
            return carry

        lax.fori_loop(0, CH_HALF // NBUF, _group, 0)
        for t in range(NBUF):
            pltpu.make_async_copy(rows.at[t],
                                  acc.at[dst_v.at[CH_HALF - NBUF + t]],
                                  ss[t]).wait()

    with jax.named_scope("ph_out"):
        plsc.subcore_barrier()

        pltpu.sync_copy(acc.at[pl.ds(base, PER_TILE)],
                        out_hbm.at[pl.ds(cid * N_PAD + base, PER_TILE)])


def _sc_agg(x, srcm, dstm):
    mesh = plsc.VectorSubcoreMesh(core_axis_name="c", subcore_axis_name="s",
                                  num_cores=2, num_subcores=16)
    return pl.kernel(
        _sc_agg_body,
        out_type=jax.ShapeDtypeStruct((2 * N_PAD, D), jnp.float32),
        mesh=mesh,
        scratch_types=[
            pltpu.VMEM((CH_HALF, CHUNK), jnp.int32),
            pltpu.VMEM((CH_HALF, CHUNK), jnp.int32),
            pltpu.VMEM((NBUF, CHUNK, D), jnp.float32),
            pltpu.VMEM_SHARED((N_PAD, D), jnp.float32),
            pltpu.SemaphoreType.DMA,
            pltpu.SemaphoreType.DMA,
            pltpu.SemaphoreType.DMA,
            pltpu.SemaphoreType.DMA,
        ],
    )(x, srcm, dstm)


def _mlp_conv_body(x_ref, p0_ref, p1_ref, wa_ref, ba_ref, wb_ref, bb_ref,
                   o_ref):
    z = x_ref[...] + p0_ref[...] + p1_ref[...]
    z = jnp.maximum(
        jnp.dot(z, wa_ref[...], preferred_element_type=jnp.float32)
        + ba_ref[...], 0.0)
    z = jnp.dot(z, wb_ref[...], preferred_element_type=jnp.float32) + bb_ref[...]
    o_ref[...] = jnp.maximum(z, 0.0)


def _mlp_head_body(x_ref, p0_ref, p1_ref, wa_ref, ba_ref, wb_ref, bb_ref,
                   wf1_ref, bf1_ref, wf2_ref, bf2_ref, o_ref):
    z = x_ref[...] + p0_ref[...] + p1_ref[...]
    z = jnp.maximum(
        jnp.dot(z, wa_ref[...], preferred_element_type=jnp.float32)
        + ba_ref[...], 0.0)
    z = jnp.dot(z, wb_ref[...], preferred_element_type=jnp.float32) + bb_ref[...]
    z = jnp.maximum(z, 0.0)
    z = jnp.maximum(
        jnp.dot(z, wf1_ref[...], preferred_element_type=jnp.float32)
        + bf1_ref[...], 0.0)
    o_ref[...] = (jnp.dot(z, wf2_ref[...], preferred_element_type=jnp.float32)
                  + bf2_ref[...])


def _row_spec():
    return pl.BlockSpec((ROW_BLK, D), lambda i: (i, 0))


def _p_specs():
    off = N_PAD // ROW_BLK
    return (pl.BlockSpec((ROW_BLK, D), lambda i: (i, 0)),
            pl.BlockSpec((ROW_BLK, D), lambda i, _o=off: (i + _o, 0)))


def _w_spec():
    return pl.BlockSpec((D, D), lambda i: (0, 0))


def _b_spec():
    return pl.BlockSpec((1, D), lambda i: (0, 0))


def _mlp_conv(x, parts, wa, ba, wb, bb):
    p0s, p1s = _p_specs()
    return pl.pallas_call(
        _mlp_conv_body,
        grid=(N_PAD // ROW_BLK,),
        in_specs=[_row_spec(), p0s, p1s, _w_spec(), _b_spec(), _w_spec(),
                  _b_spec()],
        out_specs=_row_spec(),
        out_shape=jax.ShapeDtypeStruct((N_PAD, D), jnp.float32),
    )(x, parts, parts, wa, ba.reshape(1, D), wb, bb.reshape(1, D))


def _mlp_head(x, parts, wa, ba, wb, bb, wf1, bf1, wf2p, bf2p):
    p0s, p1s = _p_specs()
    return pl.pallas_call(
        _mlp_head_body,
        grid=(N_PAD // ROW_BLK,),
        in_specs=[_row_spec(), p0s, p1s, _w_spec(), _b_spec(), _w_spec(),
                  _b_spec(), _w_spec(), _b_spec(), _w_spec(), _b_spec()],
        out_specs=_row_spec(),
        out_shape=jax.ShapeDtypeStruct((N_PAD, D), jnp.float32),
    )(x, parts, parts, wa, ba.reshape(1, D), wb, bb.reshape(1, D),
      wf1, bf1.reshape(1, D), wf2p, bf2p.reshape(1, D))


def kernel(h, bf, edge_index, edge_weight, W1a, b1a, W1b, b1b, W2a, b2a,
           W2b, b2b, Wf1, bf1, Wf2, bf2):
    src = edge_index[0].astype(jnp.int32)
    dst = edge_index[1].astype(jnp.int32)
    pad = E_PAD - E
    srcm = jnp.concatenate([src, jnp.zeros((pad,), jnp.int32)]
                           ).reshape(E_PAD // CHUNK, CHUNK)
    dummy = N + (jnp.arange(pad, dtype=jnp.int32) % (N_PAD - N))
    dstm = jnp.concatenate([dst, dummy]).reshape(E_PAD // CHUNK, CHUNK)

    wf2p = jnp.pad(Wf2, ((0, 0), (0, D - 2)))
    bf2p = jnp.pad(bf2, (0, D - 2))

    h_pad = jnp.pad(h, ((0, N_PAD - N), (0, 0)))
    parts1 = _sc_agg(h, srcm, dstm)
    x1 = _mlp_conv(h_pad, parts1, W1a, b1a, W1b, b1b)
    parts2 = _sc_agg(x1, srcm, dstm)
    out_pad = _mlp_head(x1, parts2, W2a, b2a, W2b, b2b, Wf1, bf1, wf2p, bf2p)
    return out_pad[:N, :2]
